# baseline (speedup 1.0000x reference)
import functools

import jax
import jax.numpy as jnp
from jax import lax
from jax.experimental import pallas as pl
from jax.experimental.pallas import tpu as pltpu
from jax.experimental.pallas import tpu_sc as plsc

N_USERS = 50000
N_ITEMS = 50000
N_NODES = N_USERS + N_ITEMS
N_EDGES = 1600000
DIM = 32
BATCH = 4096

_INFO = plsc.get_sparse_core_info()
NC = _INFO.num_cores
NS = _INFO.num_subcores
LANES = _INFO.num_lanes

HALF = N_NODES // NC
HALFP = 50176
N_PADROWS = HALFP - HALF
STRIPE = HALFP // NS
NODESP = NC * HALFP

EROW = 128
EK = 2
CHUNK = EK * EROW
GRP6 = 6 * CHUNK

E_PAD = 1646592
E_PART = E_PAD // 32
PCH = 3216
PGROUPS = PCH // LANES
N_PCH = E_PART // PCH
SCAP = PCH + EROW
SLOT = 55296

BT = BATCH // (NC * NS)

_mesh = plsc.VectorSubcoreMesh(core_axis_name="c", subcore_axis_name="s")
_cparams = pltpu.CompilerParams(needs_layout_passes=False,
                                use_tc_tiling_on_sc=False)


def _iota16():
    return lax.iota(jnp.int32, LANES)


@functools.partial(
    pl.kernel,
    out_type=(
        jax.ShapeDtypeStruct((64 * SLOT,), jnp.int32),
        jax.ShapeDtypeStruct((64 * SLOT,), jnp.int32),
        jax.ShapeDtypeStruct((64 * SLOT,), jnp.float32),
        jax.ShapeDtypeStruct((64 * LANES,), jnp.int32),
    ),
    mesh=_mesh,
    compiler_params=_cparams,
    scratch_types=(
        [pltpu.VMEM((PCH,), jnp.int32)] * 2
        + [pltpu.VMEM((PCH,), jnp.float32)]
        + [pltpu.VMEM((SCAP,), jnp.int32)] * 8
        + [pltpu.VMEM((SCAP,), jnp.float32)] * 4
        + [pltpu.VMEM((EROW,), jnp.int32)] * 2
        + [pltpu.VMEM((EROW,), jnp.float32)]
        + [pltpu.VMEM((LANES,), jnp.int32)]
        + [pltpu.SemaphoreType.DMA] * 2
    ),
)
def _partition(srcf, dstf, wf, psrc, pdstl, pw, counts, *scr):
    in_src, in_dst = scr[0:2]
    in_w = scr[2]
    stg = [[(scr[3 + 2 * pb + b], scr[7 + 2 * pb + b], scr[11 + 2 * pb + b])
            for b in range(2)] for pb in range(2)]
    nsrc_v, ndst_v, nw_v = scr[15], scr[16], scr[17]
    cnt_v = scr[18]
    sem_in, sem_f = scr[19], scr[20]

    c = lax.axis_index("c")
    s = lax.axis_index("s")
    pid = s * NC + c
    iota = _iota16()
    ibase = pid * E_PART

    for g in range(EROW // LANES):
        gv = g * LANES + iota
        nsrc_v[pl.ds(g * LANES, LANES)] = (gv * 37 + pid * 613) & 32767
        ndst_v[pl.ds(g * LANES, LANES)] = HALF + (gv & 127)
        nw_v[pl.ds(g * LANES, LANES)] = jnp.zeros((LANES,), jnp.float32)

    off = [jnp.int32(0 * SLOT + 0), (jnp.int32(32 * SLOT))]
    off = [off[0] + pid * SLOT, off[1] + pid * SLOT]
    base = [off[0], off[1]]
    nf_prev = [jnp.int32(0), jnp.int32(0)]

    def drain_flushes(n):
        def body(i, carry):
            pltpu.make_async_copy(psrc.at[pl.ds(0, EROW)], nsrc_v,
                                  sem_f).wait()
            return carry
        lax.fori_loop(0, n, body, 0)

    for ch in range(N_PCH):
        pb = ch & 1
        drain_flushes(nf_prev[pb])

        cbase = pl.multiple_of(ibase + ch * PCH, 16)
        cp1 = pltpu.make_async_copy(srcf.at[pl.ds(cbase, PCH)], in_src, sem_in)
        cp2 = pltpu.make_async_copy(dstf.at[pl.ds(cbase, PCH)], in_dst, sem_in)
        cp3 = pltpu.make_async_copy(wf.at[pl.ds(cbase, PCH)], in_w, sem_in)
        cp1.start(); cp2.start(); cp3.start()
        cp1.wait(); cp2.wait(); cp3.wait()

        def grp_body(g, fills):
            f0, f1 = fills
            o0 = g * LANES
            dv = in_dst[pl.ds(o0, LANES)]
            sv = in_src[pl.ds(o0, LANES)]
            wv = in_w[pl.ds(o0, LANES)]
            m0 = dv < HALF
            m1 = jnp.logical_not(m0)
            plsc.store_compressed(stg[pb][0][0].at[pl.ds(f0, LANES)], sv,
                                  mask=m0)
            plsc.store_compressed(stg[pb][0][1].at[pl.ds(f0, LANES)], dv,
                                  mask=m0)
            plsc.store_compressed(stg[pb][0][2].at[pl.ds(f0, LANES)], wv,
                                  mask=m0)
            plsc.store_compressed(stg[pb][1][0].at[pl.ds(f1, LANES)], sv,
                                  mask=m1)
            plsc.store_compressed(stg[pb][1][1].at[pl.ds(f1, LANES)],
                                  dv - HALF, mask=m1)
            plsc.store_compressed(stg[pb][1][2].at[pl.ds(f1, LANES)], wv,
                                  mask=m1)
            n0 = jnp.max(plsc.all_reduce_population_count(m0))
            return (f0 + n0, f1 + (LANES - n0))

        f0, f1 = lax.fori_loop(0, PGROUPS, grp_body, (jnp.int32(0),
                                                      jnp.int32(0)))

        fills = [f0, f1]
        nf_new = jnp.int32(0)
        for b in range(2):
            fb = fills[b]
            padn = (-fb) % EROW
            for g in range(EROW // LANES):
                mk = (g * LANES + iota) < padn
                plsc.store_compressed(
                    stg[pb][b][0].at[pl.ds(fb + g * LANES, LANES)],
                    nsrc_v[pl.ds(g * LANES, LANES)], mask=mk)
                plsc.store_compressed(
                    stg[pb][b][1].at[pl.ds(fb + g * LANES, LANES)],
                    ndst_v[pl.ds(g * LANES, LANES)], mask=mk)
                plsc.store_compressed(
                    stg[pb][b][2].at[pl.ds(fb + g * LANES, LANES)],
                    nw_v[pl.ds(g * LANES, LANES)], mask=mk)
            fb = fb + padn
            nblk = fb // EROW
            outs = (psrc, pdstl, pw)

            offb = pl.multiple_of(off[b], EROW)

            def flush_body(k, carry):
                for a in range(3):
                    pltpu.make_async_copy(
                        stg[pb][b][a].at[pl.ds(k * EROW, EROW)],
                        outs[a].at[pl.ds(offb + k * EROW, EROW)],
                        sem_f).start()
                return carry

            lax.fori_loop(0, nblk, flush_body, 0)
            off[b] = off[b] + fb
            nf_new = nf_new + 3 * nblk
        nf_prev[pb] = nf_new

    drain_flushes(nf_prev[0])
    drain_flushes(nf_prev[1])

    outs = (psrc, pdstl, pw)
    nulls = (nsrc_v, ndst_v, nw_v)
    for b in range(2):
        tot = off[b] - base[b]
        tot_r = jnp.maximum(((tot + GRP6 - 1) // GRP6) * GRP6, GRP6)
        gap_blocks = (tot_r - tot) // EROW

        offb = pl.multiple_of(off[b], EROW)

        def gap_body(k, carry):
            for a in range(3):
                pltpu.make_async_copy(
                    nulls[a], outs[a].at[pl.ds(offb + k * EROW, EROW)],
                    sem_f).start()
            return carry

        lax.fori_loop(0, gap_blocks, gap_body, 0)
        drain_flushes(3 * gap_blocks)

        cnt_v[...] = jnp.full((LANES,), tot_r, jnp.int32)
        pltpu.sync_copy(
            cnt_v,
            counts.at[pl.ds(pl.multiple_of((b * 32 + pid) * LANES, LANES),
                            LANES)])


@functools.partial(
    pl.kernel,
    out_type=jax.ShapeDtypeStruct((NODESP, DIM), jnp.float32),
    mesh=_mesh,
    compiler_params=_cparams,
    scratch_types=(
        [pltpu.VMEM((EK, EROW), jnp.int32)] * 6
        + [pltpu.VMEM((CHUNK,), jnp.int32)] * 6
        + [pltpu.VMEM((CHUNK,), jnp.float32)] * 6
        + [pltpu.VMEM((CHUNK, DIM), jnp.float32)] * 3
        + [pltpu.VMEM((LANES,), jnp.int32)]
        + [pltpu.VMEM_SHARED((HALFP, DIM), jnp.float32)]
        + [pltpu.SemaphoreType.DMA] * 13
    ),
)
def _layer(table, psrc2d, pdstl, pw, counts, zeros, out, *scr):
    src_v = scr[0:6]
    dst_v = scr[6:12]
    w_v = scr[12:18]
    rows_v = scr[18:21]
    cnt_v = scr[21]
    acc = scr[22]
    sem_i = scr[23:29]
    sem_g = scr[29:32]
    sem_s = scr[32:35]

    c = lax.axis_index("c")
    s = lax.axis_index("s")

    pltpu.sync_copy(zeros, acc.at[pl.ds(s * STRIPE, STRIPE)])
    plsc.subcore_barrier()

    for pslot in range(2):
        pid = 2 * s + pslot
        slot_id = c * 32 + pid
        sbase = slot_id * SLOT

        pltpu.sync_copy(
            counts.at[pl.ds(pl.multiple_of(slot_id * LANES, LANES), LANES)],
            cnt_v)
        T = jnp.max(cnt_v[...]) // CHUNK

        rbase0 = slot_id * (SLOT // EROW)

        def idx_copies(cc, r):
            ebase = pl.multiple_of(sbase + cc * CHUNK, EROW)
            rbase = pl.multiple_of(rbase0 + cc * EK, EK)
            return (
                pltpu.make_async_copy(psrc2d.at[pl.ds(rbase, EK)],
                                      src_v[r], sem_i[r]),
                pltpu.make_async_copy(pdstl.at[pl.ds(ebase, CHUNK)],
                                      dst_v[r], sem_i[r]),
                pltpu.make_async_copy(pw.at[pl.ds(ebase, CHUNK)],
                                      w_v[r], sem_i[r]),
            )

        def fire_idx(cc, r):
            for cp in idx_copies(cc, r):
                cp.start()

        def drain_idx(cc, r):
            for cp in idx_copies(cc, r):
                cp.wait()

        def gather_copies(q, r):
            return [pltpu.make_async_copy(
                table.at[src_v[r].at[k]],
                rows_v[q].at[pl.ds(k * EROW, EROW)], sem_g[q])
                for k in range(EK)]

        def scatter_copies(q, r):
            return [pltpu.make_async_copy(
                rows_v[q].at[pl.ds(k * EROW, EROW)],
                acc.at[dst_v[r].at[pl.ds(k * EROW, EROW)]], sem_s[q])
                for k in range(EK)]

        def fire_scatter(q, r):
            for k in range(EK):
                pltpu.async_copy(
                    rows_v[q].at[pl.ds(k * EROW, EROW)],
                    acc.at[dst_v[r].at[pl.ds(k * EROW, EROW)]], sem_s[q],
                    add=True)

        def compute(q, r):
            @plsc.parallel_loop(0, CHUNK, unroll=16)
            def _(j):
                w16 = plsc.load_gather(w_v[r],
                                       [jnp.full((LANES,), j, jnp.int32)])
                r0 = rows_v[q][j, pl.ds(0, LANES)]
                r1 = rows_v[q][j, pl.ds(LANES, LANES)]
                rows_v[q][j, pl.ds(0, LANES)] = r0 * w16
                rows_v[q][j, pl.ds(LANES, LANES)] = r1 * w16

        fire_idx(0, 0)
        fire_idx(1, 1)
        drain_idx(0, 0)
        for cp in gather_copies(0, 0):
            cp.start()
        drain_idx(1, 1)
        for cp in gather_copies(1, 1):
            cp.start()
        fire_idx(2, 2)
        fire_idx(3, 3)

        def iter_body(i, carry):
            C = i * 6
            for x in range(6):
                cc = C + x
                q = x % 3
                r = x % 6
                for cp in gather_copies(q, r):
                    cp.wait()
                compute(q, r)
                fire_scatter(q, r)
                qp = (x - 1) % 3
                rp = (x - 1) % 6

                def retire():
                    for cp in scatter_copies(qp, rp):
                        cp.wait()

                if x == 0:
                    @pl.when(i > 0)
                    def _():
                        retire()
                else:
                    retire()
                cg = jnp.minimum(cc + 2, T - 1)
                rg = (x + 2) % 6
                drain_idx(cg, rg)
                for cp in gather_copies((x + 2) % 3, rg):
                    cp.start()
                fire_idx(jnp.minimum(cc + 4, T - 1), (x + 4) % 6)
            return carry

        lax.fori_loop(0, T // 6, iter_body, 0)

        for cp in scatter_copies(2, 5):
            cp.wait()
        for cp in gather_copies(0, 0):
            cp.wait()
        for cp in gather_copies(1, 1):
            cp.wait()
        drain_idx(T - 1, 2)
        drain_idx(T - 1, 3)

    plsc.subcore_barrier()

    pltpu.sync_copy(acc.at[pl.ds(s * STRIPE, STRIPE)],
                    out.at[pl.ds(c * HALFP + s * STRIPE, STRIPE)])


@functools.partial(
    pl.kernel,
    out_type=jax.ShapeDtypeStruct((BATCH,), jnp.float32),
    mesh=_mesh,
    compiler_params=_cparams,
    scratch_types=[
        pltpu.VMEM((BT,), jnp.int32),
        pltpu.VMEM((BT,), jnp.int32),
        pltpu.VMEM((4 * BT, DIM), jnp.float32),
        pltpu.VMEM((4 * BT, DIM), jnp.float32),
        pltpu.VMEM((BT * DIM,), jnp.float32),
        pltpu.VMEM((BT,), jnp.float32),
        pltpu.SemaphoreType.DMA,
        pltpu.SemaphoreType.DMA,
    ],
)
def _final(e0, e1, e2, e3, users, items, out,
           u_v, i_v, ur_v, ir_v, p_v, o_v, sem_i, sem_g):
    c = lax.axis_index("c")
    s = lax.axis_index("s")
    wid = s * NC + c
    base = wid * BT
    iota = _iota16()

    cp_u = pltpu.async_copy(users.at[pl.ds(base, BT)], u_v, sem_i)
    cp_i = pltpu.async_copy(items.at[pl.ds(base, BT)], i_v, sem_i)
    cp_u.wait()
    cp_i.wait()

    for g in range(BT // LANES):
        i_v[pl.ds(g * LANES, LANES)] = i_v[pl.ds(g * LANES, LANES)] + HALFP

    gathers = []
    for t, tab in enumerate((e0, e1, e2, e3)):
        gathers.append(pltpu.async_copy(
            tab.at[u_v], ur_v.at[pl.ds(t * BT, BT)], sem_g))
        gathers.append(pltpu.async_copy(
            tab.at[i_v], ir_v.at[pl.ds(t * BT, BT)], sem_g))
    for g in gathers:
        g.wait()

    def sum_body(j, carry):
        uacc0 = jnp.zeros((LANES,), jnp.float32)
        uacc1 = jnp.zeros((LANES,), jnp.float32)
        iacc0 = jnp.zeros((LANES,), jnp.float32)
        iacc1 = jnp.zeros((LANES,), jnp.float32)
        for t in range(4):
            uacc0 = uacc0 + ur_v[t * BT + j, pl.ds(0, LANES)]
            uacc1 = uacc1 + ur_v[t * BT + j, pl.ds(LANES, LANES)]
            iacc0 = iacc0 + ir_v[t * BT + j, pl.ds(0, LANES)]
            iacc1 = iacc1 + ir_v[t * BT + j, pl.ds(LANES, LANES)]
        p_v[pl.ds(j * DIM, LANES)] = uacc0 * iacc0
        p_v[pl.ds(j * DIM + LANES, LANES)] = uacc1 * iacc1
        return carry

    lax.fori_loop(0, BT, sum_body, 0)

    for g in range(BT // LANES):
        acc = jnp.zeros((LANES,), jnp.float32)
        rowr = (g * LANES + iota) * DIM
        for d in range(DIM):
            acc = acc + plsc.load_gather(p_v, [rowr + d])
        o_v[pl.ds(g * LANES, LANES)] = acc * (1.0 / 16.0)

    pltpu.sync_copy(o_v, out.at[pl.ds(base, BT)])


def kernel(user_table, item_table, edge_weight, edge_index, users, items):
    halfpad = jnp.zeros((N_PADROWS, DIM), jnp.float32)
    table0 = jnp.concatenate([user_table, halfpad, item_table, halfpad],
                             axis=0)

    src = edge_index[0].astype(jnp.int32)
    dst = edge_index[1].astype(jnp.int32)
    w = edge_weight.astype(jnp.float32)

    src = src + jnp.where(src >= HALF, N_PADROWS, 0).astype(jnp.int32)

    n_pad = E_PAD - N_EDGES
    pad_iota = lax.iota(jnp.int32, n_pad)
    srcf = jnp.concatenate([src, pad_iota % N_NODES])
    dstf = jnp.concatenate([dst, N_NODES + (pad_iota & 127)])
    wf = jnp.concatenate([w, jnp.zeros((n_pad,), jnp.float32)])

    zeros = jnp.zeros((STRIPE, DIM), jnp.float32)

    psrc, pdstl, pw, counts = _partition(srcf, dstf, wf)
    psrc2d = psrc.reshape(-1, EROW)

    e0 = table0
    e1 = _layer(e0, psrc2d, pdstl, pw, counts, zeros)
    e2 = _layer(e1, psrc2d, pdstl, pw, counts, zeros)
    e3 = _layer(e2, psrc2d, pdstl, pw, counts, zeros)

    return _final(e0, e1, e2, e3, users.astype(jnp.int32),
                  items.astype(jnp.int32))


# speedup vs baseline: 1.0190x; 1.0190x over previous
import functools

import jax
import jax.numpy as jnp
from jax import lax
from jax.experimental import pallas as pl
from jax.experimental.pallas import tpu as pltpu
from jax.experimental.pallas import tpu_sc as plsc

N_USERS = 50000
N_ITEMS = 50000
N_NODES = N_USERS + N_ITEMS
N_EDGES = 1600000
DIM = 32
BATCH = 4096

_INFO = plsc.get_sparse_core_info()
NC = _INFO.num_cores
NS = _INFO.num_subcores
LANES = _INFO.num_lanes

HALF = N_NODES // NC
HALFP = 50176
N_PADROWS = HALFP - HALF
STRIPE = HALFP // NS
NODESP = NC * HALFP

EROW = 128
EK = 2
CHUNK = EK * EROW
GRP6 = 6 * CHUNK

E_PAD = 1646592
E_PART = E_PAD // 32
PCH = 3216
PGROUPS = PCH // LANES
N_PCH = E_PART // PCH
SCAP = PCH + EROW
SLOT = 55296

BT = BATCH // (NC * NS)

_mesh = plsc.VectorSubcoreMesh(core_axis_name="c", subcore_axis_name="s")
_cparams = pltpu.CompilerParams(needs_layout_passes=False,
                                use_tc_tiling_on_sc=False)


def _iota16():
    return lax.iota(jnp.int32, LANES)


@functools.partial(
    pl.kernel,
    out_type=(
        jax.ShapeDtypeStruct((64 * SLOT,), jnp.int32),
        jax.ShapeDtypeStruct((64 * SLOT,), jnp.int32),
        jax.ShapeDtypeStruct((64 * SLOT,), jnp.float32),
        jax.ShapeDtypeStruct((64 * LANES,), jnp.int32),
    ),
    mesh=_mesh,
    compiler_params=_cparams,
    scratch_types=(
        [pltpu.VMEM((PCH,), jnp.int32)] * 2
        + [pltpu.VMEM((PCH,), jnp.float32)]
        + [pltpu.VMEM((SCAP,), jnp.int32)] * 8
        + [pltpu.VMEM((SCAP,), jnp.float32)] * 4
        + [pltpu.VMEM((EROW,), jnp.int32)] * 2
        + [pltpu.VMEM((EROW,), jnp.float32)]
        + [pltpu.VMEM((LANES,), jnp.int32)]
        + [pltpu.SemaphoreType.DMA] * 2
    ),
)
def _partition(srcf, dstf, wf, psrc, pdstl, pw, counts, *scr):
    in_src, in_dst = scr[0:2]
    in_w = scr[2]
    stg = [[(scr[3 + 2 * pb + b], scr[7 + 2 * pb + b], scr[11 + 2 * pb + b])
            for b in range(2)] for pb in range(2)]
    nsrc_v, ndst_v, nw_v = scr[15], scr[16], scr[17]
    cnt_v = scr[18]
    sem_in, sem_f = scr[19], scr[20]

    c = lax.axis_index("c")
    s = lax.axis_index("s")
    pid = s * NC + c
    iota = _iota16()
    ibase = pid * E_PART

    for g in range(EROW // LANES):
        gv = g * LANES + iota
        nsrc_v[pl.ds(g * LANES, LANES)] = (gv * 37 + pid * 613) & 32767
        ndst_v[pl.ds(g * LANES, LANES)] = HALF + (gv & 127)
        nw_v[pl.ds(g * LANES, LANES)] = jnp.zeros((LANES,), jnp.float32)

    off = [jnp.int32(0 * SLOT + 0), (jnp.int32(32 * SLOT))]
    off = [off[0] + pid * SLOT, off[1] + pid * SLOT]
    base = [off[0], off[1]]
    nf_prev = [jnp.int32(0), jnp.int32(0)]

    def drain_flushes(n):
        def body(i, carry):
            pltpu.make_async_copy(psrc.at[pl.ds(0, EROW)], nsrc_v,
                                  sem_f).wait()
            return carry
        lax.fori_loop(0, n, body, 0)

    for ch in range(N_PCH):
        pb = ch & 1
        drain_flushes(nf_prev[pb])

        cbase = pl.multiple_of(ibase + ch * PCH, 16)
        cp1 = pltpu.make_async_copy(srcf.at[pl.ds(cbase, PCH)], in_src, sem_in)
        cp2 = pltpu.make_async_copy(dstf.at[pl.ds(cbase, PCH)], in_dst, sem_in)
        cp3 = pltpu.make_async_copy(wf.at[pl.ds(cbase, PCH)], in_w, sem_in)
        cp1.start(); cp2.start(); cp3.start()
        cp1.wait(); cp2.wait(); cp3.wait()

        def grp_body(g, fills):
            f0, f1 = fills
            o0 = g * LANES
            dv = in_dst[pl.ds(o0, LANES)]
            sv = in_src[pl.ds(o0, LANES)]
            wv = in_w[pl.ds(o0, LANES)]
            m0 = dv < HALF
            m1 = jnp.logical_not(m0)
            plsc.store_compressed(stg[pb][0][0].at[pl.ds(f0, LANES)], sv,
                                  mask=m0)
            plsc.store_compressed(stg[pb][0][1].at[pl.ds(f0, LANES)], dv,
                                  mask=m0)
            plsc.store_compressed(stg[pb][0][2].at[pl.ds(f0, LANES)], wv,
                                  mask=m0)
            plsc.store_compressed(stg[pb][1][0].at[pl.ds(f1, LANES)], sv,
                                  mask=m1)
            plsc.store_compressed(stg[pb][1][1].at[pl.ds(f1, LANES)],
                                  dv - HALF, mask=m1)
            plsc.store_compressed(stg[pb][1][2].at[pl.ds(f1, LANES)], wv,
                                  mask=m1)
            n0 = jnp.max(plsc.all_reduce_population_count(m0))
            return (f0 + n0, f1 + (LANES - n0))

        f0, f1 = lax.fori_loop(0, PGROUPS, grp_body, (jnp.int32(0),
                                                      jnp.int32(0)))

        fills = [f0, f1]
        nf_new = jnp.int32(0)
        for b in range(2):
            fb = fills[b]
            padn = (-fb) % EROW
            for g in range(EROW // LANES):
                mk = (g * LANES + iota) < padn
                plsc.store_compressed(
                    stg[pb][b][0].at[pl.ds(fb + g * LANES, LANES)],
                    nsrc_v[pl.ds(g * LANES, LANES)], mask=mk)
                plsc.store_compressed(
                    stg[pb][b][1].at[pl.ds(fb + g * LANES, LANES)],
                    ndst_v[pl.ds(g * LANES, LANES)], mask=mk)
                plsc.store_compressed(
                    stg[pb][b][2].at[pl.ds(fb + g * LANES, LANES)],
                    nw_v[pl.ds(g * LANES, LANES)], mask=mk)
            fb = fb + padn
            nblk = fb // EROW
            outs = (psrc, pdstl, pw)

            offb = pl.multiple_of(off[b], EROW)

            def flush_body(k, carry):
                for a in range(3):
                    pltpu.make_async_copy(
                        stg[pb][b][a].at[pl.ds(k * EROW, EROW)],
                        outs[a].at[pl.ds(offb + k * EROW, EROW)],
                        sem_f).start()
                return carry

            lax.fori_loop(0, nblk, flush_body, 0)
            off[b] = off[b] + fb
            nf_new = nf_new + 3 * nblk
        nf_prev[pb] = nf_new

    drain_flushes(nf_prev[0])
    drain_flushes(nf_prev[1])

    outs = (psrc, pdstl, pw)
    nulls = (nsrc_v, ndst_v, nw_v)
    for b in range(2):
        tot = off[b] - base[b]
        tot_r = jnp.maximum(((tot + GRP6 - 1) // GRP6) * GRP6, GRP6)
        gap_blocks = (tot_r - tot) // EROW

        offb = pl.multiple_of(off[b], EROW)

        def gap_body(k, carry):
            for a in range(3):
                pltpu.make_async_copy(
                    nulls[a], outs[a].at[pl.ds(offb + k * EROW, EROW)],
                    sem_f).start()
            return carry

        lax.fori_loop(0, gap_blocks, gap_body, 0)
        drain_flushes(3 * gap_blocks)

        cnt_v[...] = jnp.full((LANES,), tot_r, jnp.int32)
        pltpu.sync_copy(
            cnt_v,
            counts.at[pl.ds(pl.multiple_of((b * 32 + pid) * LANES, LANES),
                            LANES)])


@functools.partial(
    pl.kernel,
    out_type=jax.ShapeDtypeStruct((NODESP, DIM), jnp.float32),
    mesh=_mesh,
    compiler_params=_cparams,
    scratch_types=(
        [pltpu.VMEM((EK, EROW), jnp.int32)] * 6
        + [pltpu.VMEM((CHUNK,), jnp.int32)] * 6
        + [pltpu.VMEM((CHUNK,), jnp.float32)] * 6
        + [pltpu.VMEM((CHUNK, DIM), jnp.float32)] * 3
        + [pltpu.VMEM((LANES,), jnp.int32)]
        + [pltpu.VMEM_SHARED((HALFP, DIM), jnp.float32)]
        + [pltpu.SemaphoreType.DMA] * 13
    ),
)
def _layer(table, psrc2d, pdstl, pw, counts, zeros, out, *scr):
    src_v = scr[0:6]
    dst_v = scr[6:12]
    w_v = scr[12:18]
    rows_v = scr[18:21]
    cnt_v = scr[21]
    acc = scr[22]
    sem_i = scr[23:29]
    sem_g = scr[29:32]
    sem_s = scr[32:35]

    c = lax.axis_index("c")
    s = lax.axis_index("s")

    pltpu.sync_copy(zeros, acc.at[pl.ds(s * STRIPE, STRIPE)])
    plsc.subcore_barrier()

    if True:
        slot1 = c * 32 + 2 * s
        slot2 = slot1 + 1

        pltpu.sync_copy(
            counts.at[pl.ds(pl.multiple_of(slot1 * LANES, LANES), LANES)],
            cnt_v)
        T1 = jnp.max(cnt_v[...]) // CHUNK
        pltpu.sync_copy(
            counts.at[pl.ds(pl.multiple_of(slot2 * LANES, LANES), LANES)],
            cnt_v)
        T2 = jnp.max(cnt_v[...]) // CHUNK
        T = T1 + T2

        sb1 = slot1 * SLOT
        sb2 = slot2 * SLOT
        rb1 = slot1 * (SLOT // EROW)
        rb2 = slot2 * (SLOT // EROW)

        def idx_copies(cc, r):
            in2 = cc >= T1
            ebase = pl.multiple_of(
                jnp.where(in2, sb2 + (cc - T1) * CHUNK, sb1 + cc * CHUNK),
                EROW)
            rbase = pl.multiple_of(
                jnp.where(in2, rb2 + (cc - T1) * EK, rb1 + cc * EK), EK)
            return (
                pltpu.make_async_copy(psrc2d.at[pl.ds(rbase, EK)],
                                      src_v[r], sem_i[r]),
                pltpu.make_async_copy(pdstl.at[pl.ds(ebase, CHUNK)],
                                      dst_v[r], sem_i[r]),
                pltpu.make_async_copy(pw.at[pl.ds(ebase, CHUNK)],
                                      w_v[r], sem_i[r]),
            )

        def fire_idx(cc, r):
            for cp in idx_copies(cc, r):
                cp.start()

        def drain_idx(cc, r):
            for cp in idx_copies(cc, r):
                cp.wait()

        def gather_copies(q, r):
            return [pltpu.make_async_copy(
                table.at[src_v[r].at[k]],
                rows_v[q].at[pl.ds(k * EROW, EROW)], sem_g[q])
                for k in range(EK)]

        def scatter_copies(q, r):
            return [pltpu.make_async_copy(
                rows_v[q].at[pl.ds(k * EROW, EROW)],
                acc.at[dst_v[r].at[pl.ds(k * EROW, EROW)]], sem_s[q])
                for k in range(EK)]

        def fire_scatter(q, r):
            for k in range(EK):
                pltpu.async_copy(
                    rows_v[q].at[pl.ds(k * EROW, EROW)],
                    acc.at[dst_v[r].at[pl.ds(k * EROW, EROW)]], sem_s[q],
                    add=True)

        def compute(q, r):
            @plsc.parallel_loop(0, CHUNK, unroll=8)
            def _(j):
                w16 = plsc.load_gather(w_v[r],
                                       [jnp.full((LANES,), j, jnp.int32)])
                r0 = rows_v[q][j, pl.ds(0, LANES)]
                r1 = rows_v[q][j, pl.ds(LANES, LANES)]
                rows_v[q][j, pl.ds(0, LANES)] = r0 * w16
                rows_v[q][j, pl.ds(LANES, LANES)] = r1 * w16

        fire_idx(0, 0)
        fire_idx(1, 1)
        drain_idx(0, 0)
        for cp in gather_copies(0, 0):
            cp.start()
        drain_idx(1, 1)
        for cp in gather_copies(1, 1):
            cp.start()
        fire_idx(2, 2)
        fire_idx(3, 3)

        def iter_body(i, carry):
            C = i * 6
            for x in range(6):
                cc = C + x
                q = x % 3
                r = x % 6
                for cp in gather_copies(q, r):
                    cp.wait()
                compute(q, r)
                fire_scatter(q, r)
                qp = (x - 1) % 3
                rp = (x - 1) % 6

                def retire():
                    for cp in scatter_copies(qp, rp):
                        cp.wait()

                if x == 0:
                    @pl.when(i > 0)
                    def _():
                        retire()
                else:
                    retire()
                cg = jnp.minimum(cc + 2, T - 1)
                rg = (x + 2) % 6
                drain_idx(cg, rg)
                for cp in gather_copies((x + 2) % 3, rg):
                    cp.start()
                fire_idx(jnp.minimum(cc + 4, T - 1), (x + 4) % 6)
            return carry

        lax.fori_loop(0, T // 6, iter_body, 0)

        for cp in scatter_copies(2, 5):
            cp.wait()
        for cp in gather_copies(0, 0):
            cp.wait()
        for cp in gather_copies(1, 1):
            cp.wait()
        drain_idx(T - 1, 2)
        drain_idx(T - 1, 3)

    plsc.subcore_barrier()

    pltpu.sync_copy(acc.at[pl.ds(s * STRIPE, STRIPE)],
                    out.at[pl.ds(c * HALFP + s * STRIPE, STRIPE)])


@functools.partial(
    pl.kernel,
    out_type=jax.ShapeDtypeStruct((BATCH,), jnp.float32),
    mesh=_mesh,
    compiler_params=_cparams,
    scratch_types=[
        pltpu.VMEM((BT,), jnp.int32),
        pltpu.VMEM((BT,), jnp.int32),
        pltpu.VMEM((4 * BT, DIM), jnp.float32),
        pltpu.VMEM((4 * BT, DIM), jnp.float32),
        pltpu.VMEM((BT * DIM,), jnp.float32),
        pltpu.VMEM((BT,), jnp.float32),
        pltpu.SemaphoreType.DMA,
        pltpu.SemaphoreType.DMA,
    ],
)
def _final(e0, e1, e2, e3, users, items, out,
           u_v, i_v, ur_v, ir_v, p_v, o_v, sem_i, sem_g):
    c = lax.axis_index("c")
    s = lax.axis_index("s")
    wid = s * NC + c
    base = wid * BT
    iota = _iota16()

    cp_u = pltpu.async_copy(users.at[pl.ds(base, BT)], u_v, sem_i)
    cp_i = pltpu.async_copy(items.at[pl.ds(base, BT)], i_v, sem_i)
    cp_u.wait()
    cp_i.wait()

    for g in range(BT // LANES):
        i_v[pl.ds(g * LANES, LANES)] = i_v[pl.ds(g * LANES, LANES)] + HALFP

    gathers = []
    for t, tab in enumerate((e0, e1, e2, e3)):
        gathers.append(pltpu.async_copy(
            tab.at[u_v], ur_v.at[pl.ds(t * BT, BT)], sem_g))
        gathers.append(pltpu.async_copy(
            tab.at[i_v], ir_v.at[pl.ds(t * BT, BT)], sem_g))
    for g in gathers:
        g.wait()

    def sum_body(j, carry):
        uacc0 = jnp.zeros((LANES,), jnp.float32)
        uacc1 = jnp.zeros((LANES,), jnp.float32)
        iacc0 = jnp.zeros((LANES,), jnp.float32)
        iacc1 = jnp.zeros((LANES,), jnp.float32)
        for t in range(4):
            uacc0 = uacc0 + ur_v[t * BT + j, pl.ds(0, LANES)]
            uacc1 = uacc1 + ur_v[t * BT + j, pl.ds(LANES, LANES)]
            iacc0 = iacc0 + ir_v[t * BT + j, pl.ds(0, LANES)]
            iacc1 = iacc1 + ir_v[t * BT + j, pl.ds(LANES, LANES)]
        p_v[pl.ds(j * DIM, LANES)] = uacc0 * iacc0
        p_v[pl.ds(j * DIM + LANES, LANES)] = uacc1 * iacc1
        return carry

    lax.fori_loop(0, BT, sum_body, 0)

    for g in range(BT // LANES):
        acc = jnp.zeros((LANES,), jnp.float32)
        rowr = (g * LANES + iota) * DIM
        for d in range(DIM):
            acc = acc + plsc.load_gather(p_v, [rowr + d])
        o_v[pl.ds(g * LANES, LANES)] = acc * (1.0 / 16.0)

    pltpu.sync_copy(o_v, out.at[pl.ds(base, BT)])


def kernel(user_table, item_table, edge_weight, edge_index, users, items):
    halfpad = jnp.zeros((N_PADROWS, DIM), jnp.float32)
    table0 = jnp.concatenate([user_table, halfpad, item_table, halfpad],
                             axis=0)

    src = edge_index[0].astype(jnp.int32)
    dst = edge_index[1].astype(jnp.int32)
    w = edge_weight.astype(jnp.float32)

    src = src + jnp.where(src >= HALF, N_PADROWS, 0).astype(jnp.int32)

    n_pad = E_PAD - N_EDGES
    pad_iota = lax.iota(jnp.int32, n_pad)
    srcf = jnp.concatenate([src, pad_iota % N_NODES])
    dstf = jnp.concatenate([dst, N_NODES + (pad_iota & 127)])
    wf = jnp.concatenate([w, jnp.zeros((n_pad,), jnp.float32)])

    zeros = jnp.zeros((STRIPE, DIM), jnp.float32)

    psrc, pdstl, pw, counts = _partition(srcf, dstf, wf)
    psrc2d = psrc.reshape(-1, EROW)

    e0 = table0
    e1 = _layer(e0, psrc2d, pdstl, pw, counts, zeros)
    e2 = _layer(e1, psrc2d, pdstl, pw, counts, zeros)
    e3 = _layer(e2, psrc2d, pdstl, pw, counts, zeros)

    return _final(e0, e1, e2, e3, users.astype(jnp.int32),
                  items.astype(jnp.int32))
